# baseline (speedup 1.0000x reference)
import functools

import jax
import jax.numpy as jnp
from jax import lax
from jax.experimental import pallas as pl
from jax.experimental.pallas import tpu as pltpu
from jax.experimental.pallas import tpu_sc as plsc

EPS = 1e-05
N = 100000
D = 120
N_SCALAR = 32
N_VEC = 16
N_TEN = 8
NW = 32
CHUNK = 160
NCHUNKS = N // CHUNK
TILES = CHUNK // 16


def _rsqrt_newton(x):
    i = lax.bitcast_convert_type(x, jnp.int32)
    i = jnp.int32(0x5F3759DF) - lax.shift_right_arithmetic(i, 1)
    y = lax.bitcast_convert_type(i, jnp.float32)
    y = y * (1.5 - 0.5 * x * y * y)
    y = y * (1.5 - 0.5 * x * y * y)
    return y


def _make_sc_kernel():
    mesh = plsc.VectorSubcoreMesh(core_axis_name="c", subcore_axis_name="s")
    info = plsc.get_sparse_core_info()
    nc = info.num_cores

    @functools.partial(
        pl.kernel,
        mesh=mesh,
        out_type=jax.ShapeDtypeStruct((N, D), jnp.float32),
        scratch_types=[
            pltpu.VMEM((CHUNK, D), jnp.float32),
            pltpu.VMEM((CHUNK, D), jnp.float32),
            pltpu.VMEM((CHUNK, D), jnp.float32),
            pltpu.VMEM((CHUNK, D), jnp.float32),
            pltpu.VMEM((N_SCALAR, 16), jnp.float32),
            pltpu.VMEM((N_SCALAR, 16), jnp.float32),
            pltpu.SemaphoreType.DMA,
            pltpu.SemaphoreType.DMA,
            pltpu.SemaphoreType.DMA,
            pltpu.SemaphoreType.DMA,
        ],
        compiler_params=pltpu.CompilerParams(needs_layout_passes=False),
    )
    def sc_eq_ln(x_hbm, w_hbm, b_hbm, out_hbm,
                 ib0, ib1, ob0, ob1, wv, bv, sin0, sin1, sout0, sout1):
        wid = lax.axis_index("s") * nc + lax.axis_index("c")
        n_i = NCHUNKS // NW + jnp.where(wid < NCHUNKS % NW, 1, 0)
        ibs, obs = (ib0, ib1), (ob0, ob1)
        sins, souts = (sin0, sin1), (sout0, sout1)

        pltpu.sync_copy(w_hbm, wv)
        pltpu.sync_copy(b_hbm, bv)

        def row_base(i):
            return (wid + i * NW) * CHUNK

        pltpu.async_copy(x_hbm.at[pl.ds(row_base(0), CHUNK)], ib0, sin0)

        def compute_chunk(ib, ob):
            def tile_body(t, carry):
                rid = t * 16 + lax.iota(jnp.int32, 16)

                def col(cc):
                    return jnp.full((16,), cc, jnp.int32)

                vs = [plsc.load_gather(ib, [rid, col(c)])
                      for c in range(N_SCALAR)]
                acc = vs[0]
                acc2 = vs[0] * vs[0]
                for c in range(1, N_SCALAR):
                    acc = acc + vs[c]
                    acc2 = acc2 + vs[c] * vs[c]
                m = acc * (1.0 / N_SCALAR)
                var = acc2 * (1.0 / N_SCALAR) - m * m
                inv = _rsqrt_newton(var + EPS)
                for c in range(N_SCALAR):
                    y = (vs[c] - m) * inv * wv[c, :] + bv[c, :]
                    plsc.store_scatter(ob, [rid, col(c)], y)

                for g in range(N_VEC):
                    c0 = N_SCALAR + 3 * g
                    a = plsc.load_gather(ib, [rid, col(c0)])
                    b = plsc.load_gather(ib, [rid, col(c0 + 1)])
                    cq = plsc.load_gather(ib, [rid, col(c0 + 2)])
                    ss = a * a + b * b + cq * cq
                    ir = _rsqrt_newton(ss * (1.0 / 3.0) + EPS)
                    plsc.store_scatter(ob, [rid, col(c0)], a * ir)
                    plsc.store_scatter(ob, [rid, col(c0 + 1)], b * ir)
                    plsc.store_scatter(ob, [rid, col(c0 + 2)], cq * ir)

                for g in range(N_TEN):
                    c0 = N_SCALAR + 3 * N_VEC + 5 * g
                    es = [plsc.load_gather(ib, [rid, col(c0 + j)])
                          for j in range(5)]
                    ss = es[0] * es[0]
                    for j in range(1, 5):
                        ss = ss + es[j] * es[j]
                    ir = _rsqrt_newton(ss * (1.0 / 5.0) + EPS)
                    for j in range(5):
                        plsc.store_scatter(ob, [rid, col(c0 + j)], es[j] * ir)
                return carry

            lax.fori_loop(0, TILES, tile_body, 0)

        def pair_body(s, carry):
            for b in range(2):
                i = s * 2 + b

                @pl.when(i < n_i)
                def _():
                    @pl.when(i + 1 < n_i)
                    def _():
                        pltpu.async_copy(
                            x_hbm.at[pl.ds(row_base(i + 1), CHUNK)],
                            ibs[1 - b], sins[1 - b])

                    pltpu.make_async_copy(
                        x_hbm.at[pl.ds(row_base(i), CHUNK)],
                        ibs[b], sins[b]).wait()

                    @pl.when(i >= 2)
                    def _():
                        pltpu.make_async_copy(
                            obs[b],
                            out_hbm.at[pl.ds(row_base(i - 2), CHUNK)],
                            souts[b]).wait()

                    compute_chunk(ibs[b], obs[b])

                    pltpu.async_copy(
                        obs[b],
                        out_hbm.at[pl.ds(row_base(i), CHUNK)],
                        souts[b])
            return carry

        lax.fori_loop(0, (NCHUNKS // NW + 2) // 2, pair_body, 0)

        for b in range(2):
            for back in (2, 1):
                i_d = n_i - back

                @pl.when((i_d >= 0) & (lax.rem(i_d, 2) == b))
                def _():
                    pltpu.make_async_copy(
                        obs[b],
                        out_hbm.at[pl.ds(row_base(i_d), CHUNK)],
                        souts[b]).wait()

    return sc_eq_ln


_SC_KERNEL = _make_sc_kernel()


def kernel(x, weight, bias):
    w16 = jnp.broadcast_to(weight[:, None], (N_SCALAR, 16)).astype(jnp.float32)
    b16 = jnp.broadcast_to(bias[:, None], (N_SCALAR, 16)).astype(jnp.float32)
    return _SC_KERNEL(x, w16, b16)


# speedup vs baseline: 4.5331x; 4.5331x over previous
import jax
import jax.numpy as jnp
import numpy as np
from jax.experimental import pallas as pl

EPS = 1e-05
N_SCALAR = 32
N_VEC = 16
N_TEN = 8
D = N_SCALAR + 3 * N_VEC + 5 * N_TEN

BLOCK_ROWS = 20000


def _group_maps():
    m = np.zeros((D, D), dtype=np.float32)
    m[:N_SCALAR, :N_SCALAR] = 1.0 / N_SCALAR
    g = np.zeros((D, D), dtype=np.float32)
    g[:N_SCALAR, :N_SCALAR] = 1.0 / N_SCALAR
    off = N_SCALAR
    for _ in range(N_VEC):
        g[off:off + 3, off:off + 3] = 1.0
        off += 3
    for _ in range(N_TEN):
        g[off:off + 5, off:off + 5] = 1.0
        off += 5
    k = np.concatenate([
        np.ones(N_SCALAR, np.float32),
        np.full(3 * N_VEC, 3.0, np.float32),
        np.full(5 * N_TEN, 5.0, np.float32),
    ])
    keps = (k * EPS).reshape(1, D)
    sqrtk = np.sqrt(k).reshape(1, D)
    return m, g, keps, sqrtk


_M_NP, _G_NP, _KEPS_NP, _SQRTK_NP = _group_maps()


def _eq_ln_kernel(x_ref, mm_ref, gg_ref, keps_ref, w_ref, b_ref, o_ref):
    x = x_ref[...]
    xb = x.astype(jnp.bfloat16)
    sqb = (x * x).astype(jnp.bfloat16)
    m = jnp.dot(xb, mm_ref[...], preferred_element_type=jnp.float32)
    msq = jnp.dot(sqb, gg_ref[...], preferred_element_type=jnp.float32)
    denom = msq - m * m + keps_ref[...]
    y = (x - m) * jax.lax.rsqrt(denom)
    o_ref[...] = y * w_ref[...] + b_ref[...]


def kernel(x, weight, bias):
    n, d = x.shape
    mm = jnp.asarray(_M_NP, dtype=jnp.bfloat16)
    gg = jnp.asarray(_G_NP, dtype=jnp.bfloat16)
    keps = jnp.asarray(_KEPS_NP)
    w_full = (jnp.concatenate([weight, jnp.ones((d - N_SCALAR,), x.dtype)])
              .reshape(1, d) * jnp.asarray(_SQRTK_NP))
    b_full = jnp.concatenate([bias, jnp.zeros((d - N_SCALAR,), x.dtype)]).reshape(1, d)
    grid = (n // BLOCK_ROWS,)
    return pl.pallas_call(
        _eq_ln_kernel,
        grid=grid,
        in_specs=[
            pl.BlockSpec((BLOCK_ROWS, d), lambda i: (i, 0)),
            pl.BlockSpec((d, d), lambda i: (0, 0)),
            pl.BlockSpec((d, d), lambda i: (0, 0)),
            pl.BlockSpec((1, d), lambda i: (0, 0)),
            pl.BlockSpec((1, d), lambda i: (0, 0)),
            pl.BlockSpec((1, d), lambda i: (0, 0)),
        ],
        out_specs=pl.BlockSpec((BLOCK_ROWS, d), lambda i: (i, 0)),
        out_shape=jax.ShapeDtypeStruct((n, d), x.dtype),
    )(x, mm, gg, keps, w_full, b_full)
